# baseline (speedup 1.0000x reference)
import functools

import jax
import jax.numpy as jnp
from jax import lax
from jax.experimental import pallas as pl
from jax.experimental.pallas import tpu as pltpu

_TH = 16


def _gram_kernel(feat_ref, out_ref, *, nsteps, scale, half):
    kk = pl.program_id(0)

    @pl.when(kk == 0)
    def _():
        out_ref[...] = jnp.zeros_like(out_ref)

    f = feat_ref[...].astype(jnp.bfloat16)
    flat = f.reshape(f.shape[0], f.shape[1] * f.shape[2])
    top = flat[:half]
    bot = flat[half:]

    out_ref[:half, :] += lax.dot_general(
        top, flat,
        dimension_numbers=(((1,), (1,)), ((), ())),
        preferred_element_type=jnp.float32,
    )
    out_ref[half:, half:] += lax.dot_general(
        bot, bot,
        dimension_numbers=(((1,), (1,)), ((), ())),
        preferred_element_type=jnp.float32,
    )

    @pl.when(kk == nsteps - 1)
    def _():
        out_ref[:half, :] = out_ref[:half, :] * scale
        out_ref[half:, half:] = out_ref[half:, half:] * scale
        out_ref[half:, :half] = jnp.swapaxes(out_ref[:half, half:], 0, 1)


def kernel(x):
    b, c, h, w = x.shape
    m = b * c
    feats = x.reshape(m, h, w)
    scale = 1.0 / float(b * c * h * w)

    th = _TH
    while th > 1 and h % th:
        th //= 2
    steps = h // th

    return pl.pallas_call(
        functools.partial(_gram_kernel, nsteps=steps, scale=scale, half=m // 2),
        out_shape=jax.ShapeDtypeStruct((m, m), jnp.float32),
        grid=(steps,),
        in_specs=[pl.BlockSpec((m, th, w), lambda kk: (0, kk, 0))],
        out_specs=pl.BlockSpec((m, m), lambda kk: (0, 0)),
        compiler_params=pltpu.CompilerParams(
            dimension_semantics=("arbitrary",),
            vmem_limit_bytes=64 << 20,
        ),
    )(feats)
